# baseline (speedup 1.0000x reference)
import functools

import jax
import jax.numpy as jnp
from jax import lax
from jax.experimental import pallas as pl
from jax.experimental.pallas import tpu as pltpu
from jax.experimental.pallas import tpu_sc as plsc

N = 10000
NP = 10240
E = 320000
G = 128
DIN = 128
DHID = 64
NCLS = 4
NC = 2
NS = 16
NW = NC * NS
CH = 128
NCH = 80
EP = NW * NCH * CH
NCHT = 2 * NCH
NCH0 = 64
NCH1 = NCHT - NCH0
PH = 32
NBUF = 4
DAG = 64
ZR = NP // NS
BN = 1024
NB = NP // BN


def _sc_mesh():
    return plsc.VectorSubcoreMesh(
        core_axis_name="c", subcore_axis_name="s",
        num_cores=NC, num_subcores=NS)


def _make_count_kernel():
    D = 16

    @functools.partial(
        pl.kernel,
        out_type=jax.ShapeDtypeStruct((NC, NP, D), jnp.float32),
        mesh=_sc_mesh(),
        scratch_types=[
            pltpu.VMEM((NCH, CH), jnp.int32),
            pltpu.VMEM((CH, D), jnp.float32),
            pltpu.VMEM_SHARED((NP, D), jnp.float32),
        ],
        compiler_params=pltpu.CompilerParams(use_tc_tiling_on_sc=False),
    )
    def cnt_kernel(dst_hbm, out_hbm, idx_dst, rowbuf, acc):
        c = lax.axis_index("c")
        s = lax.axis_index("s")
        w = c * NS + s
        pltpu.sync_copy(dst_hbm.at[w], idx_dst)

        def zrow(r, carry):
            rowbuf[r, :] = jnp.zeros((16,), jnp.float32)
            return carry
        lax.fori_loop(0, CH, zrow, 0)
        for k in range(ZR // CH):
            pltpu.sync_copy(rowbuf, acc.at[pl.ds(s * ZR + k * CH, CH)])

        def orow(r, carry):
            rowbuf[r, :] = jnp.full((16,), 1.0, jnp.float32)
            return carry
        lax.fori_loop(0, CH, orow, 0)
        plsc.subcore_barrier()

        def body(j, carry):
            pltpu.sync_copy(rowbuf, acc.at[idx_dst.at[j]], add=True)
            return carry
        lax.fori_loop(0, NCH, body, 0)
        plsc.subcore_barrier()
        pltpu.sync_copy(acc.at[pl.ds(s * ZR, ZR)],
                        out_hbm.at[c, pl.ds(s * ZR, ZR)])

    return cnt_kernel


def _make_agg_kernel(H):
    D = DAG

    @functools.partial(
        pl.kernel,
        out_type=jax.ShapeDtypeStruct((NC, H, NP, D), jnp.float32),
        mesh=_sc_mesh(),
        scratch_types=[
            pltpu.VMEM((PH, CH), jnp.int32),
            pltpu.VMEM((PH, CH), jnp.int32),
            pltpu.VMEM((NBUF, CH, D), jnp.float32),
            pltpu.VMEM_SHARED((NP, D), jnp.float32),
            pltpu.VMEM_SHARED((NP, D), jnp.float32),
            pltpu.SemaphoreType.DMA,
        ],
        compiler_params=pltpu.CompilerParams(use_tc_tiling_on_sc=False),
    )
    def agg_kernel(hs_hbm, src_hbm, dst_hbm, out_hbm,
                   idx_src, idx_dst, rows, acc, hs_sp, gsem):
        c = lax.axis_index("c")
        s = lax.axis_index("s")
        base = jnp.where(c == 0, 0, NCH0)
        nphase = jnp.where(c == 0, NCH0 // PH, NCH1 // PH)
        nz = D // 16

        for h in range(H):
            def zrow(r, carry):
                def zcol(cc, carry2):
                    rows[0, r, pl.ds(cc * 16, 16)] = (
                        jnp.zeros((16,), jnp.float32))
                    return carry2
                return lax.fori_loop(0, nz, zcol, carry)
            lax.fori_loop(0, CH, zrow, 0)
            for k in range(ZR // CH):
                pltpu.sync_copy(rows.at[0], acc.at[pl.ds(s * ZR + k * CH, CH)])

            @pl.when(c == 1)
            def _stage():
                pltpu.sync_copy(hs_hbm.at[h, pl.ds(s * ZR, ZR)],
                                hs_sp.at[pl.ds(s * ZR, ZR)])
            plsc.subcore_barrier()

            def run_loop(gsrc):
                def phase(ph, carry):
                    pltpu.sync_copy(src_hbm.at[s, pl.ds(base + ph * PH, PH)],
                                    idx_src)
                    pltpu.sync_copy(dst_hbm.at[s, pl.ds(base + ph * PH, PH)],
                                    idx_dst)
                    for q in range(NBUF - 1):
                        pltpu.async_copy(gsrc.at[idx_src.at[q]],
                                         rows.at[q], gsem)

                    def body(j, carry2):
                        jb = lax.rem(j, NBUF)
                        pltpu.make_async_copy(gsrc.at[idx_src.at[j]],
                                              rows.at[jb], gsem).wait()

                        @pl.when(j < PH - (NBUF - 1))
                        def _prefetch():
                            pltpu.async_copy(
                                gsrc.at[idx_src.at[j + NBUF - 1]],
                                rows.at[lax.rem(j + NBUF - 1, NBUF)], gsem)

                        pltpu.sync_copy(rows.at[jb], acc.at[idx_dst.at[j]],
                                        add=True)
                        return carry2
                    lax.fori_loop(0, PH, body, 0)
                    return carry
                lax.fori_loop(0, nphase, phase, 0)

            @pl.when(c == 0)
            def _c0():
                run_loop(hs_hbm.at[h])

            @pl.when(c == 1)
            def _c1():
                run_loop(hs_sp)
            plsc.subcore_barrier()
            pltpu.sync_copy(acc.at[pl.ds(s * ZR, ZR)],
                            out_hbm.at[c, h, pl.ds(s * ZR, ZR)])

    return agg_kernel


_count_edges = _make_count_kernel()
_agg_h2 = _make_agg_kernel(2)
_agg_h1 = _make_agg_kernel(1)


def _k1a_body(x_ref, w_ref, hw_ref):
    hw_ref[...] = jnp.dot(x_ref[...], w_ref[...],
                          preferred_element_type=jnp.float32)


_k1a = pl.pallas_call(
    _k1a_body,
    grid=(NB,),
    in_specs=[
        pl.BlockSpec((BN, DIN), lambda i: (i, 0)),
        pl.BlockSpec((DIN, DIN), lambda i: (0, 0)),
    ],
    out_specs=pl.BlockSpec((BN, DIN), lambda i: (i, 0)),
    out_shape=jax.ShapeDtypeStruct((NP, DIN), jnp.float32),
)


def _k1b_body(cnt_ref, hw_ref, hs_ref, dinv_ref):
    ct = cnt_ref[...]
    csum = ct[0] + ct[1]
    deg = jnp.sum(csum, axis=1) * (1.0 / 16.0) + 1.0
    dinv = lax.rsqrt(deg)
    hw = hw_ref[...]
    hs_ref[0] = hw[:, :DAG] * dinv[:, None]
    hs_ref[1] = hw[:, DAG:] * dinv[:, None]
    dinv_ref[0, 0, :] = dinv


_k1b = pl.pallas_call(
    _k1b_body,
    grid=(NB,),
    in_specs=[
        pl.BlockSpec((NC, BN, 16), lambda i: (0, i, 0)),
        pl.BlockSpec((BN, DIN), lambda i: (i, 0)),
    ],
    out_specs=[
        pl.BlockSpec((2, BN, DAG), lambda i: (0, i, 0)),
        pl.BlockSpec((1, 1, BN), lambda i: (i, 0, 0)),
    ],
    out_shape=[
        jax.ShapeDtypeStruct((2, NP, DAG), jnp.float32),
        jax.ShapeDtypeStruct((NB, 1, BN), jnp.float32),
    ],
)


def _k3_body(agg_ref, hs1_ref, dinv_ref, b_ref, w_ref, hs2_ref):
    dinv = dinv_ref[0, 0, :]
    agg = jnp.concatenate(
        [agg_ref[0, 0] + agg_ref[1, 0], agg_ref[0, 1] + agg_ref[1, 1]],
        axis=1)
    m = agg + jnp.concatenate([hs1_ref[0], hs1_ref[1]], axis=1)
    h1 = jnp.maximum(dinv[:, None] * m + b_ref[...], 0.0)
    hw = jnp.dot(h1, w_ref[...], preferred_element_type=jnp.float32)
    hs2_ref[...] = hw * dinv[:, None]


_k3 = pl.pallas_call(
    _k3_body,
    grid=(NB,),
    in_specs=[
        pl.BlockSpec((NC, 2, BN, DAG), lambda i: (0, 0, i, 0)),
        pl.BlockSpec((2, BN, DAG), lambda i: (0, i, 0)),
        pl.BlockSpec((1, 1, BN), lambda i: (i, 0, 0)),
        pl.BlockSpec((1, DIN), lambda i: (0, 0)),
        pl.BlockSpec((DIN, DHID), lambda i: (0, 0)),
    ],
    out_specs=pl.BlockSpec((BN, DHID), lambda i: (i, 0)),
    out_shape=jax.ShapeDtypeStruct((NP, DHID), jnp.float32),
)


def _k5_body(agg_ref, hs2_ref, dinv_ref, bg_ref, batch_ref,
             w1_ref, b1_ref, w2_ref, b2_ref, out_ref, sums_scr, cnt_scr):
    i = pl.program_id(0)

    @pl.when(i == 0)
    def _init():
        sums_scr[...] = jnp.zeros_like(sums_scr)
        cnt_scr[...] = jnp.zeros_like(cnt_scr)

    dinv = dinv_ref[0, 0, :]
    m = agg_ref[0, 0] + agg_ref[1, 0] + hs2_ref[...]
    h2 = jnp.maximum(dinv[:, None] * m + bg_ref[...], 0.0)
    b = batch_ref[0, 0, :]
    gids = lax.broadcasted_iota(jnp.int32, (G, BN), 0)
    oh = (gids == b[None, :]).astype(jnp.float32)
    sums_scr[...] += jnp.dot(oh, h2, preferred_element_type=jnp.float32)
    cnt_scr[...] += jnp.sum(oh, axis=1)[:, None]

    @pl.when(i == NB - 1)
    def _final():
        pooled = sums_scr[...] / jnp.maximum(cnt_scr[...], 1.0)
        z = jnp.maximum(
            jnp.dot(pooled, w1_ref[...], preferred_element_type=jnp.float32)
            + b1_ref[...], 0.0)
        out_ref[...] = (
            jnp.dot(z, w2_ref[...], preferred_element_type=jnp.float32)
            + b2_ref[...])


_k5 = pl.pallas_call(
    _k5_body,
    grid=(NB,),
    in_specs=[
        pl.BlockSpec((NC, 1, BN, DHID), lambda i: (0, 0, i, 0)),
        pl.BlockSpec((BN, DHID), lambda i: (i, 0)),
        pl.BlockSpec((1, 1, BN), lambda i: (i, 0, 0)),
        pl.BlockSpec((1, DHID), lambda i: (0, 0)),
        pl.BlockSpec((1, 1, BN), lambda i: (i, 0, 0)),
        pl.BlockSpec((DHID, 32), lambda i: (0, 0)),
        pl.BlockSpec((1, 32), lambda i: (0, 0)),
        pl.BlockSpec((32, NCLS), lambda i: (0, 0)),
        pl.BlockSpec((1, NCLS), lambda i: (0, 0)),
    ],
    out_specs=pl.BlockSpec((G, NCLS), lambda i: (0, 0)),
    out_shape=jax.ShapeDtypeStruct((G, NCLS), jnp.float32),
    scratch_shapes=[
        pltpu.VMEM((G, DHID), jnp.float32),
        pltpu.VMEM((G, DHID), jnp.float32),
    ],
)


def kernel(x, edge_index, batch, W_enc, b_enc, W_gcn, b_gcn, W1, b1, W2, b2):
    src = edge_index[0].astype(jnp.int32)
    dst = edge_index[1].astype(jnp.int32)
    pad = EP - E
    src_f = jnp.concatenate([src, jnp.zeros((pad,), jnp.int32)])
    trash = N + jnp.arange(pad, dtype=jnp.int32) % (NP - N)
    dst_f = jnp.concatenate([dst, trash])
    dst_p = dst_f.reshape(NW, NCH, CH)
    src_a = src_f.reshape(NS, NCHT, CH)
    dst_a = dst_f.reshape(NS, NCHT, CH)
    x_p = jnp.pad(x, ((0, NP - N), (0, 0)))
    batch_p = jnp.pad(batch.astype(jnp.int32), (0, NP - N),
                      constant_values=G).reshape(NB, 1, BN)

    counts = _count_edges(dst_p)
    hw1 = _k1a(x_p, W_enc)
    hs1_h, dinv3 = _k1b(counts, hw1)
    agg1 = _agg_h2(hs1_h, src_a, dst_a)
    hs2 = _k3(agg1, hs1_h, dinv3, b_enc.reshape(1, DIN), W_gcn)
    agg2 = _agg_h1(hs2[None], src_a, dst_a)
    logits = _k5(agg2, hs2, dinv3, b_gcn.reshape(1, DHID), batch_p,
                 W1, b1.reshape(1, 32), W2, b2.reshape(1, NCLS))
    return logits


# speedup vs baseline: 1.1179x; 1.1179x over previous
import functools

import jax
import jax.numpy as jnp
from jax import lax
from jax.experimental import pallas as pl
from jax.experimental.pallas import tpu as pltpu
from jax.experimental.pallas import tpu_sc as plsc

N = 10000
NP = 10240
E = 320000
G = 128
DIN = 128
DHID = 64
NCLS = 4
NC = 2
NS = 16
NW = NC * NS
CH = 128
NCH = 80
EP = NW * NCH * CH
NCHT = 2 * NCH
NCH0 = 80
NCH1 = NCHT - NCH0
PH = 40
NBUF = 4
DAG = 64
ZR = NP // NS
BN = 1024
NB = NP // BN


def _sc_mesh():
    return plsc.VectorSubcoreMesh(
        core_axis_name="c", subcore_axis_name="s",
        num_cores=NC, num_subcores=NS)


def _make_count_kernel():
    D = 16

    @functools.partial(
        pl.kernel,
        out_type=jax.ShapeDtypeStruct((NC, NP, D), jnp.float32),
        mesh=_sc_mesh(),
        scratch_types=[
            pltpu.VMEM((NCH, CH), jnp.int32),
            pltpu.VMEM((CH, D), jnp.float32),
            pltpu.VMEM_SHARED((NP, D), jnp.float32),
        ],
        compiler_params=pltpu.CompilerParams(use_tc_tiling_on_sc=False),
    )
    def cnt_kernel(dst_hbm, out_hbm, idx_dst, rowbuf, acc):
        c = lax.axis_index("c")
        s = lax.axis_index("s")
        w = c * NS + s
        pltpu.sync_copy(dst_hbm.at[w], idx_dst)

        def zrow(r, carry):
            rowbuf[r, :] = jnp.zeros((16,), jnp.float32)
            return carry
        lax.fori_loop(0, CH, zrow, 0)
        for k in range(ZR // CH):
            pltpu.sync_copy(rowbuf, acc.at[pl.ds(s * ZR + k * CH, CH)])

        def orow(r, carry):
            rowbuf[r, :] = jnp.full((16,), 1.0, jnp.float32)
            return carry
        lax.fori_loop(0, CH, orow, 0)
        plsc.subcore_barrier()

        def body(j, carry):
            pltpu.sync_copy(rowbuf, acc.at[idx_dst.at[j]], add=True)
            return carry
        lax.fori_loop(0, NCH, body, 0)
        plsc.subcore_barrier()
        pltpu.sync_copy(acc.at[pl.ds(s * ZR, ZR)],
                        out_hbm.at[c, pl.ds(s * ZR, ZR)])

    return cnt_kernel


def _make_agg_kernel(H):
    D = DAG

    @functools.partial(
        pl.kernel,
        out_type=jax.ShapeDtypeStruct((NC, H, NP, D), jnp.float32),
        mesh=_sc_mesh(),
        scratch_types=[
            pltpu.VMEM((PH, CH), jnp.int32),
            pltpu.VMEM((PH, CH), jnp.int32),
            pltpu.VMEM((NBUF, CH, D), jnp.float32),
            pltpu.VMEM_SHARED((NP, D), jnp.float32),
            pltpu.VMEM_SHARED((NP, D), jnp.float32),
            pltpu.SemaphoreType.DMA,
        ],
        compiler_params=pltpu.CompilerParams(use_tc_tiling_on_sc=False),
    )
    def agg_kernel(hs_hbm, src_hbm, dst_hbm, out_hbm,
                   idx_src, idx_dst, rows, acc, hs_sp, gsem):
        c = lax.axis_index("c")
        s = lax.axis_index("s")
        base = jnp.where(c == 0, 0, NCH0)
        nphase = jnp.where(c == 0, NCH0 // PH, NCH1 // PH)
        nz = D // 16

        for h in range(H):
            def zrow(r, carry):
                def zcol(cc, carry2):
                    rows[0, r, pl.ds(cc * 16, 16)] = (
                        jnp.zeros((16,), jnp.float32))
                    return carry2
                return lax.fori_loop(0, nz, zcol, carry)
            lax.fori_loop(0, CH, zrow, 0)
            for k in range(ZR // CH):
                pltpu.sync_copy(rows.at[0], acc.at[pl.ds(s * ZR + k * CH, CH)])

            pltpu.sync_copy(hs_hbm.at[h, pl.ds(s * ZR, ZR)],
                            hs_sp.at[pl.ds(s * ZR, ZR)])
            plsc.subcore_barrier()

            def run_loop(gsrc):
                def phase(ph, carry):
                    pltpu.sync_copy(src_hbm.at[s, pl.ds(base + ph * PH, PH)],
                                    idx_src)
                    pltpu.sync_copy(dst_hbm.at[s, pl.ds(base + ph * PH, PH)],
                                    idx_dst)
                    for q in range(NBUF - 1):
                        pltpu.async_copy(gsrc.at[idx_src.at[q]],
                                         rows.at[q], gsem)

                    def body(j, carry2):
                        jb = lax.rem(j, NBUF)
                        pltpu.make_async_copy(gsrc.at[idx_src.at[j]],
                                              rows.at[jb], gsem).wait()

                        @pl.when(j < PH - (NBUF - 1))
                        def _prefetch():
                            pltpu.async_copy(
                                gsrc.at[idx_src.at[j + NBUF - 1]],
                                rows.at[lax.rem(j + NBUF - 1, NBUF)], gsem)

                        pltpu.sync_copy(rows.at[jb], acc.at[idx_dst.at[j]],
                                        add=True)
                        return carry2
                    lax.fori_loop(0, PH, body, 0)
                    return carry
                lax.fori_loop(0, nphase, phase, 0)

            run_loop(hs_sp)
            plsc.subcore_barrier()
            pltpu.sync_copy(acc.at[pl.ds(s * ZR, ZR)],
                            out_hbm.at[c, h, pl.ds(s * ZR, ZR)])

    return agg_kernel


_count_edges = _make_count_kernel()
_agg_h2 = _make_agg_kernel(2)
_agg_h1 = _make_agg_kernel(1)


def _k1a_body(x_ref, w_ref, hw_ref):
    hw_ref[...] = jnp.dot(x_ref[...], w_ref[...],
                          preferred_element_type=jnp.float32)


_k1a = pl.pallas_call(
    _k1a_body,
    grid=(NB,),
    in_specs=[
        pl.BlockSpec((BN, DIN), lambda i: (i, 0)),
        pl.BlockSpec((DIN, DIN), lambda i: (0, 0)),
    ],
    out_specs=pl.BlockSpec((BN, DIN), lambda i: (i, 0)),
    out_shape=jax.ShapeDtypeStruct((NP, DIN), jnp.float32),
)


def _k1b_body(cnt_ref, hw_ref, hs_ref, dinv_ref):
    ct = cnt_ref[...]
    csum = ct[0] + ct[1]
    deg = jnp.sum(csum, axis=1) * (1.0 / 16.0) + 1.0
    dinv = lax.rsqrt(deg)
    hw = hw_ref[...]
    hs_ref[0] = hw[:, :DAG] * dinv[:, None]
    hs_ref[1] = hw[:, DAG:] * dinv[:, None]
    dinv_ref[0, 0, :] = dinv


_k1b = pl.pallas_call(
    _k1b_body,
    grid=(NB,),
    in_specs=[
        pl.BlockSpec((NC, BN, 16), lambda i: (0, i, 0)),
        pl.BlockSpec((BN, DIN), lambda i: (i, 0)),
    ],
    out_specs=[
        pl.BlockSpec((2, BN, DAG), lambda i: (0, i, 0)),
        pl.BlockSpec((1, 1, BN), lambda i: (i, 0, 0)),
    ],
    out_shape=[
        jax.ShapeDtypeStruct((2, NP, DAG), jnp.float32),
        jax.ShapeDtypeStruct((NB, 1, BN), jnp.float32),
    ],
)


def _k3_body(agg_ref, hs1_ref, dinv_ref, b_ref, w_ref, hs2_ref):
    dinv = dinv_ref[0, 0, :]
    agg = jnp.concatenate(
        [agg_ref[0, 0] + agg_ref[1, 0], agg_ref[0, 1] + agg_ref[1, 1]],
        axis=1)
    m = agg + jnp.concatenate([hs1_ref[0], hs1_ref[1]], axis=1)
    h1 = jnp.maximum(dinv[:, None] * m + b_ref[...], 0.0)
    hw = jnp.dot(h1, w_ref[...], preferred_element_type=jnp.float32)
    hs2_ref[...] = hw * dinv[:, None]


_k3 = pl.pallas_call(
    _k3_body,
    grid=(NB,),
    in_specs=[
        pl.BlockSpec((NC, 2, BN, DAG), lambda i: (0, 0, i, 0)),
        pl.BlockSpec((2, BN, DAG), lambda i: (0, i, 0)),
        pl.BlockSpec((1, 1, BN), lambda i: (i, 0, 0)),
        pl.BlockSpec((1, DIN), lambda i: (0, 0)),
        pl.BlockSpec((DIN, DHID), lambda i: (0, 0)),
    ],
    out_specs=pl.BlockSpec((BN, DHID), lambda i: (i, 0)),
    out_shape=jax.ShapeDtypeStruct((NP, DHID), jnp.float32),
)


def _k5_body(agg_ref, hs2_ref, dinv_ref, bg_ref, batch_ref,
             w1_ref, b1_ref, w2_ref, b2_ref, out_ref, sums_scr, cnt_scr):
    i = pl.program_id(0)

    @pl.when(i == 0)
    def _init():
        sums_scr[...] = jnp.zeros_like(sums_scr)
        cnt_scr[...] = jnp.zeros_like(cnt_scr)

    dinv = dinv_ref[0, 0, :]
    m = agg_ref[0, 0] + agg_ref[1, 0] + hs2_ref[...]
    h2 = jnp.maximum(dinv[:, None] * m + bg_ref[...], 0.0)
    b = batch_ref[0, 0, :]
    gids = lax.broadcasted_iota(jnp.int32, (G, BN), 0)
    oh = (gids == b[None, :]).astype(jnp.float32)
    sums_scr[...] += jnp.dot(oh, h2, preferred_element_type=jnp.float32)
    cnt_scr[...] += jnp.sum(oh, axis=1)[:, None]

    @pl.when(i == NB - 1)
    def _final():
        pooled = sums_scr[...] / jnp.maximum(cnt_scr[...], 1.0)
        z = jnp.maximum(
            jnp.dot(pooled, w1_ref[...], preferred_element_type=jnp.float32)
            + b1_ref[...], 0.0)
        out_ref[...] = (
            jnp.dot(z, w2_ref[...], preferred_element_type=jnp.float32)
            + b2_ref[...])


_k5 = pl.pallas_call(
    _k5_body,
    grid=(NB,),
    in_specs=[
        pl.BlockSpec((NC, 1, BN, DHID), lambda i: (0, 0, i, 0)),
        pl.BlockSpec((BN, DHID), lambda i: (i, 0)),
        pl.BlockSpec((1, 1, BN), lambda i: (i, 0, 0)),
        pl.BlockSpec((1, DHID), lambda i: (0, 0)),
        pl.BlockSpec((1, 1, BN), lambda i: (i, 0, 0)),
        pl.BlockSpec((DHID, 32), lambda i: (0, 0)),
        pl.BlockSpec((1, 32), lambda i: (0, 0)),
        pl.BlockSpec((32, NCLS), lambda i: (0, 0)),
        pl.BlockSpec((1, NCLS), lambda i: (0, 0)),
    ],
    out_specs=pl.BlockSpec((G, NCLS), lambda i: (0, 0)),
    out_shape=jax.ShapeDtypeStruct((G, NCLS), jnp.float32),
    scratch_shapes=[
        pltpu.VMEM((G, DHID), jnp.float32),
        pltpu.VMEM((G, DHID), jnp.float32),
    ],
)


def kernel(x, edge_index, batch, W_enc, b_enc, W_gcn, b_gcn, W1, b1, W2, b2):
    src = edge_index[0].astype(jnp.int32)
    dst = edge_index[1].astype(jnp.int32)
    pad = EP - E
    src_f = jnp.concatenate([src, jnp.zeros((pad,), jnp.int32)])
    trash = N + jnp.arange(pad, dtype=jnp.int32) % (NP - N)
    dst_f = jnp.concatenate([dst, trash])
    dst_p = dst_f.reshape(NW, NCH, CH)
    src_a = src_f.reshape(NS, NCHT, CH)
    dst_a = dst_f.reshape(NS, NCHT, CH)
    x_p = jnp.pad(x, ((0, NP - N), (0, 0)))
    batch_p = jnp.pad(batch.astype(jnp.int32), (0, NP - N),
                      constant_values=G).reshape(NB, 1, BN)

    counts = _count_edges(dst_p)
    hw1 = _k1a(x_p, W_enc)
    hs1_h, dinv3 = _k1b(counts, hw1)
    agg1 = _agg_h2(hs1_h, src_a, dst_a)
    hs2 = _k3(agg1, hs1_h, dinv3, b_enc.reshape(1, DIN), W_gcn)
    agg2 = _agg_h1(hs2[None], src_a, dst_a)
    logits = _k5(agg2, hs2, dinv3, b_gcn.reshape(1, DHID), batch_p,
                 W1, b1.reshape(1, 32), W2, b2.reshape(1, NCLS))
    return logits


# speedup vs baseline: 1.2662x; 1.1326x over previous
import functools

import jax
import jax.numpy as jnp
from jax import lax
from jax.experimental import pallas as pl
from jax.experimental.pallas import tpu as pltpu
from jax.experimental.pallas import tpu_sc as plsc

N = 10000
NP = 10240
E = 320000
G = 128
DIN = 128
DHID = 64
NCLS = 4
NC = 2
NS = 16
NW = NC * NS
CH = 128
NCH = 80
EP = NW * NCH * CH
NCHT = 2 * NCH
NCH0 = 80
NCH1 = NCHT - NCH0
PH = 40
NBUF = 4
DAG = 64
ZR = NP // NS
BN = 1024
NB = NP // BN


def _sc_mesh():
    return plsc.VectorSubcoreMesh(
        core_axis_name="c", subcore_axis_name="s",
        num_cores=NC, num_subcores=NS)


def _make_count_kernel():
    D = 16

    @functools.partial(
        pl.kernel,
        out_type=jax.ShapeDtypeStruct((NC, NP, D), jnp.float32),
        mesh=_sc_mesh(),
        scratch_types=[
            pltpu.VMEM((NCH, CH), jnp.int32),
            pltpu.VMEM((CH, D), jnp.float32),
            pltpu.VMEM_SHARED((NP, D), jnp.float32),
        ],
        compiler_params=pltpu.CompilerParams(use_tc_tiling_on_sc=False),
    )
    def cnt_kernel(dst_hbm, out_hbm, idx_dst, rowbuf, acc):
        c = lax.axis_index("c")
        s = lax.axis_index("s")
        w = c * NS + s
        pltpu.sync_copy(dst_hbm.at[w], idx_dst)

        def zrow(r, carry):
            rowbuf[r, :] = jnp.zeros((16,), jnp.float32)
            return carry
        lax.fori_loop(0, CH, zrow, 0)
        for k in range(ZR // CH):
            pltpu.sync_copy(rowbuf, acc.at[pl.ds(s * ZR + k * CH, CH)])

        def orow(r, carry):
            rowbuf[r, :] = jnp.full((16,), 1.0, jnp.float32)
            return carry
        lax.fori_loop(0, CH, orow, 0)
        plsc.subcore_barrier()

        def body(j, carry):
            pltpu.sync_copy(rowbuf, acc.at[idx_dst.at[j]], add=True)
            return carry
        lax.fori_loop(0, NCH, body, 0)
        plsc.subcore_barrier()
        pltpu.sync_copy(acc.at[pl.ds(s * ZR, ZR)],
                        out_hbm.at[c, pl.ds(s * ZR, ZR)])

    return cnt_kernel


def _make_agg_kernel(H):
    D = DAG

    @functools.partial(
        pl.kernel,
        out_type=jax.ShapeDtypeStruct((NC, NP, DIN), jnp.float32),
        mesh=_sc_mesh(),
        scratch_types=[
            pltpu.VMEM((PH, CH), jnp.int32),
            pltpu.VMEM((PH, CH), jnp.int32),
            pltpu.VMEM((NBUF, CH, D), jnp.float32),
            pltpu.VMEM_SHARED((NP, D), jnp.float32),
            pltpu.VMEM_SHARED((NP, D), jnp.float32),
            pltpu.SemaphoreType.DMA,
        ],
        compiler_params=pltpu.CompilerParams(use_tc_tiling_on_sc=False),
    )
    def agg_kernel(hs_hbm, src_hbm, dst_hbm, out_hbm,
                   idx_src, idx_dst, rows, acc, hs_sp, gsem):
        c = lax.axis_index("c")
        s = lax.axis_index("s")
        base = jnp.where(c == 0, 0, NCH0)
        nphase = jnp.where(c == 0, NCH0 // PH, NCH1 // PH)
        nz = D // 16

        for h in range(H):
            def zrow(r, carry):
                def zcol(cc, carry2):
                    rows[0, r, pl.ds(cc * 16, 16)] = (
                        jnp.zeros((16,), jnp.float32))
                    return carry2
                return lax.fori_loop(0, nz, zcol, carry)
            lax.fori_loop(0, CH, zrow, 0)
            for k in range(ZR // CH):
                pltpu.sync_copy(rows.at[0], acc.at[pl.ds(s * ZR + k * CH, CH)])

            pltpu.sync_copy(hs_hbm.at[pl.ds(s * ZR, ZR), pl.ds(h * D, D)],
                            hs_sp.at[pl.ds(s * ZR, ZR)])
            plsc.subcore_barrier()

            def run_loop(gsrc):
                def phase(ph, carry):
                    pltpu.sync_copy(src_hbm.at[s, pl.ds(base + ph * PH, PH)],
                                    idx_src)
                    pltpu.sync_copy(dst_hbm.at[s, pl.ds(base + ph * PH, PH)],
                                    idx_dst)
                    for q in range(NBUF - 1):
                        pltpu.async_copy(gsrc.at[idx_src.at[q]],
                                         rows.at[q], gsem)

                    def body(j, carry2):
                        jb = lax.rem(j, NBUF)
                        pltpu.make_async_copy(gsrc.at[idx_src.at[j]],
                                              rows.at[jb], gsem).wait()

                        @pl.when(j < PH - (NBUF - 1))
                        def _prefetch():
                            pltpu.async_copy(
                                gsrc.at[idx_src.at[j + NBUF - 1]],
                                rows.at[lax.rem(j + NBUF - 1, NBUF)], gsem)

                        pltpu.sync_copy(rows.at[jb], acc.at[idx_dst.at[j]],
                                        add=True)
                        return carry2
                    lax.fori_loop(0, PH, body, 0)
                    return carry
                lax.fori_loop(0, nphase, phase, 0)

            run_loop(hs_sp)
            plsc.subcore_barrier()
            pltpu.sync_copy(acc.at[pl.ds(s * ZR, ZR)],
                            out_hbm.at[c, pl.ds(s * ZR, ZR),
                                       pl.ds(h * D, D)])

    return agg_kernel


_count_edges = _make_count_kernel()
_agg_h2 = _make_agg_kernel(2)
_agg_h1 = _make_agg_kernel(1)


def _k1a_body(x_ref, w_ref, hw_ref):
    hw_ref[...] = jnp.dot(x_ref[...], w_ref[...],
                          preferred_element_type=jnp.float32)


_k1a = pl.pallas_call(
    _k1a_body,
    grid=(NB,),
    in_specs=[
        pl.BlockSpec((BN, DIN), lambda i: (i, 0)),
        pl.BlockSpec((DIN, DIN), lambda i: (0, 0)),
    ],
    out_specs=pl.BlockSpec((BN, DIN), lambda i: (i, 0)),
    out_shape=jax.ShapeDtypeStruct((NP, DIN), jnp.float32),
)


def _k1b_body(cnt_ref, hw_ref, hs_ref, dinv_ref):
    ct = cnt_ref[...]
    csum = ct[0] + ct[1]
    deg = jnp.sum(csum, axis=1) * (1.0 / 16.0) + 1.0
    dinv = lax.rsqrt(deg)
    hs_ref[...] = hw_ref[...] * dinv[:, None]
    dinv_ref[0, 0, :] = dinv


_k1b = pl.pallas_call(
    _k1b_body,
    grid=(NB,),
    in_specs=[
        pl.BlockSpec((NC, BN, 16), lambda i: (0, i, 0)),
        pl.BlockSpec((BN, DIN), lambda i: (i, 0)),
    ],
    out_specs=[
        pl.BlockSpec((BN, DIN), lambda i: (i, 0)),
        pl.BlockSpec((1, 1, BN), lambda i: (i, 0, 0)),
    ],
    out_shape=[
        jax.ShapeDtypeStruct((NP, DIN), jnp.float32),
        jax.ShapeDtypeStruct((NB, 1, BN), jnp.float32),
    ],
)


def _k3_body(agg_ref, hs1_ref, dinv_ref, b_ref, w_ref, hs2_ref):
    dinv = dinv_ref[0, 0, :]
    m = agg_ref[0] + agg_ref[1] + hs1_ref[...]
    h1 = jnp.maximum(dinv[:, None] * m + b_ref[...], 0.0)
    hw = jnp.dot(h1, w_ref[...], preferred_element_type=jnp.float32)
    hs2_ref[...] = jnp.concatenate(
        [hw * dinv[:, None], jnp.zeros((BN, DIN - DHID), jnp.float32)],
        axis=1)


_k3 = pl.pallas_call(
    _k3_body,
    grid=(NB,),
    in_specs=[
        pl.BlockSpec((NC, BN, DIN), lambda i: (0, i, 0)),
        pl.BlockSpec((BN, DIN), lambda i: (i, 0)),
        pl.BlockSpec((1, 1, BN), lambda i: (i, 0, 0)),
        pl.BlockSpec((1, DIN), lambda i: (0, 0)),
        pl.BlockSpec((DIN, DHID), lambda i: (0, 0)),
    ],
    out_specs=pl.BlockSpec((BN, DIN), lambda i: (i, 0)),
    out_shape=jax.ShapeDtypeStruct((NP, DIN), jnp.float32),
)


def _k5_body(agg_ref, hs2_ref, dinv_ref, bg_ref, batch_ref,
             w1_ref, b1_ref, w2_ref, b2_ref, out_ref, sums_scr, cnt_scr):
    i = pl.program_id(0)

    @pl.when(i == 0)
    def _init():
        sums_scr[...] = jnp.zeros_like(sums_scr)
        cnt_scr[...] = jnp.zeros_like(cnt_scr)

    dinv = dinv_ref[0, 0, :]
    m = (agg_ref[0] + agg_ref[1] + hs2_ref[...])[:, :DHID]
    h2 = jnp.maximum(dinv[:, None] * m + bg_ref[...], 0.0)
    b = batch_ref[0, 0, :]
    gids = lax.broadcasted_iota(jnp.int32, (G, BN), 0)
    oh = (gids == b[None, :]).astype(jnp.float32)
    sums_scr[...] += jnp.dot(oh, h2, preferred_element_type=jnp.float32)
    cnt_scr[...] += jnp.sum(oh, axis=1)[:, None]

    @pl.when(i == NB - 1)
    def _final():
        pooled = sums_scr[...] / jnp.maximum(cnt_scr[...], 1.0)
        z = jnp.maximum(
            jnp.dot(pooled, w1_ref[...], preferred_element_type=jnp.float32)
            + b1_ref[...], 0.0)
        out_ref[...] = (
            jnp.dot(z, w2_ref[...], preferred_element_type=jnp.float32)
            + b2_ref[...])


_k5 = pl.pallas_call(
    _k5_body,
    grid=(NB,),
    in_specs=[
        pl.BlockSpec((NC, BN, DIN), lambda i: (0, i, 0)),
        pl.BlockSpec((BN, DIN), lambda i: (i, 0)),
        pl.BlockSpec((1, 1, BN), lambda i: (i, 0, 0)),
        pl.BlockSpec((1, DHID), lambda i: (0, 0)),
        pl.BlockSpec((1, 1, BN), lambda i: (i, 0, 0)),
        pl.BlockSpec((DHID, 32), lambda i: (0, 0)),
        pl.BlockSpec((1, 32), lambda i: (0, 0)),
        pl.BlockSpec((32, NCLS), lambda i: (0, 0)),
        pl.BlockSpec((1, NCLS), lambda i: (0, 0)),
    ],
    out_specs=pl.BlockSpec((G, NCLS), lambda i: (0, 0)),
    out_shape=jax.ShapeDtypeStruct((G, NCLS), jnp.float32),
    scratch_shapes=[
        pltpu.VMEM((G, DHID), jnp.float32),
        pltpu.VMEM((G, DHID), jnp.float32),
    ],
)


def kernel(x, edge_index, batch, W_enc, b_enc, W_gcn, b_gcn, W1, b1, W2, b2):
    src = edge_index[0].astype(jnp.int32)
    dst = edge_index[1].astype(jnp.int32)
    pad = EP - E
    src_f = jnp.concatenate([src, jnp.zeros((pad,), jnp.int32)])
    trash = N + jnp.arange(pad, dtype=jnp.int32) % (NP - N)
    dst_f = jnp.concatenate([dst, trash])
    dst_p = dst_f.reshape(NW, NCH, CH)
    src_a = src_f.reshape(NS, NCHT, CH)
    dst_a = dst_f.reshape(NS, NCHT, CH)
    x_p = jnp.pad(x, ((0, NP - N), (0, 0)))
    batch_p = jnp.pad(batch.astype(jnp.int32), (0, NP - N),
                      constant_values=G).reshape(NB, 1, BN)

    counts = _count_edges(dst_p)
    hw1 = _k1a(x_p, W_enc)
    hs1, dinv3 = _k1b(counts, hw1)
    agg1 = _agg_h2(hs1, src_a, dst_a)
    hs2 = _k3(agg1, hs1, dinv3, b_enc.reshape(1, DIN), W_gcn)
    agg2 = _agg_h1(hs2, src_a, dst_a)
    logits = _k5(agg2, hs2, dinv3, b_gcn.reshape(1, DHID), batch_p,
                 W1, b1.reshape(1, 32), W2, b2.reshape(1, NCLS))
    return logits
